# baseline (speedup 1.0000x reference)
import functools

import jax
import jax.numpy as jnp
from jax import lax
from jax.experimental import pallas as pl
from jax.experimental.pallas import tpu as pltpu
from jax.experimental.pallas import tpu_sc as plsc

NC, NS = 2, 16
NW = NC * NS
B = 32768
D = 1024
BPW = B // NW
C = 16
NCHUNK = BPW // C
NBUF = 6

_MESH = plsc.VectorSubcoreMesh(
    core_axis_name="c", subcore_axis_name="s", num_cores=NC, num_subcores=NS
)


@functools.partial(
    pl.kernel,
    out_type=jax.ShapeDtypeStruct((B, D), jnp.float32),
    mesh=_MESH,
    scratch_types=[
        pltpu.VMEM((BPW,), jnp.int32),
        [pltpu.VMEM((C, D), jnp.float32) for _ in range(NBUF)],
        [pltpu.SemaphoreType.DMA for _ in range(NBUF)],
        [pltpu.SemaphoreType.DMA for _ in range(NBUF)],
    ],
)
def _gather(pos_hbm, table_hbm, out_hbm, idx_v, rows, gsem, osem):
    wid = lax.axis_index("s") * NC + lax.axis_index("c")
    base = wid * BPW
    pltpu.sync_copy(pos_hbm.at[pl.ds(base, BPW)], idx_v)

    def start_gather(g, b):
        pltpu.async_copy(table_hbm.at[idx_v.at[pl.ds(g * C, C)]], rows[b], gsem[b])

    def wait_gather(b):
        pltpu.make_async_copy(table_hbm.at[pl.ds(0, C)], rows[b], gsem[b]).wait()

    def start_wb(g, b):
        pltpu.async_copy(rows[b], out_hbm.at[pl.ds(base + g * C, C)], osem[b])

    def wait_wb(b):
        pltpu.make_async_copy(rows[b], out_hbm.at[pl.ds(base, C)], osem[b]).wait()

    def slot(gg, b, lookahead, first_round):
        wait_gather(b)
        start_wb(gg, b)
        if lookahead:
            tbuf = (b - 1) % NBUF
            if not first_round:
                wait_wb(tbuf)
            start_gather(gg + NBUF - 1, tbuf)

    for k in range(NBUF - 1):
        start_gather(k, k)

    for gg in range(NBUF):
        slot(gg, gg % NBUF, True, gg == 0)

    last_la = NCHUNK - NBUF
    n_loop = (last_la + 1 - NBUF) // NBUF
    loop_end = NBUF + n_loop * NBUF

    @pl.loop(NBUF, loop_end, step=NBUF)
    def _rot(g):
        for ss in range(NBUF):
            slot(g + ss, ss, True, False)

    for gg in range(loop_end, last_la + 1):
        slot(gg, gg % NBUF, True, False)

    for gg in range(last_la + 1, NCHUNK):
        slot(gg, gg % NBUF, False, False)

    for g in range(NCHUNK - NBUF, NCHUNK):
        wait_wb(g % NBUF)


def kernel(position, weight):
    return _gather(position.astype(jnp.int32), weight)
